# baseline (speedup 1.0000x reference)
import jax
import jax.numpy as jnp
from jax import lax
from jax.experimental import pallas as pl
from jax.experimental.pallas import tpu as pltpu
from jax.experimental.pallas import tpu_sc as plsc

BATCH = 16384
EMBED_DIM = 64
TRUNK_DIM = 128

_NUM_CORES = 2
_NUM_SUBCORES = 16
_NW = _NUM_CORES * _NUM_SUBCORES
_B_PER_W = BATCH // _NW


def _gather_body(idx_hbm, table_hbm, out_hbm, idx_v, rows_v, sem):
  wid = lax.axis_index("s") * _NUM_CORES + lax.axis_index("c")
  base = wid * _B_PER_W
  pltpu.sync_copy(idx_hbm.at[pl.ds(base, _B_PER_W)], idx_v)
  pltpu.async_copy(table_hbm.at[idx_v], rows_v, sem).wait()
  pltpu.sync_copy(rows_v, out_hbm.at[pl.ds(base, _B_PER_W)])


@jax.jit
def _sc_gather(cls_idx, table):
  mesh = plsc.VectorSubcoreMesh(core_axis_name="c", subcore_axis_name="s")
  return pl.kernel(
      _gather_body,
      out_type=jax.ShapeDtypeStruct((BATCH, EMBED_DIM), jnp.float32),
      mesh=mesh,
      compiler_params=pltpu.CompilerParams(use_tc_tiling_on_sc=False),
      scratch_types=[
          pltpu.VMEM((_B_PER_W,), jnp.int32),
          pltpu.VMEM((_B_PER_W, EMBED_DIM), jnp.float32),
          pltpu.SemaphoreType.DMA,
      ],
  )(cls_idx, table)


_BLK = 2048


def _mlp_body(emb_ref, w1_ref, b1_ref, w2_ref, b2_ref, out_ref):
  h = jnp.dot(emb_ref[...], w1_ref[...], preferred_element_type=jnp.float32)
  h = h + b1_ref[...]
  h = h * jax.nn.sigmoid(h)
  o = jnp.dot(h, w2_ref[...], preferred_element_type=jnp.float32)
  out_ref[...] = o + b2_ref[...]


@jax.jit
def _tc_mlp(emb, W1, b1, W2, b2):
  grid = (BATCH // _BLK,)
  return pl.pallas_call(
      _mlp_body,
      grid=grid,
      in_specs=[
          pl.BlockSpec((_BLK, EMBED_DIM), lambda i: (i, 0)),
          pl.BlockSpec((EMBED_DIM, TRUNK_DIM), lambda i: (0, 0)),
          pl.BlockSpec((1, TRUNK_DIM), lambda i: (0, 0)),
          pl.BlockSpec((TRUNK_DIM, TRUNK_DIM), lambda i: (0, 0)),
          pl.BlockSpec((1, TRUNK_DIM), lambda i: (0, 0)),
      ],
      out_specs=pl.BlockSpec((_BLK, TRUNK_DIM), lambda i: (i, 0)),
      out_shape=jax.ShapeDtypeStruct((BATCH, TRUNK_DIM), jnp.float32),
  )(emb, W1, b1.reshape(1, TRUNK_DIM), W2, b2.reshape(1, TRUNK_DIM))


def kernel(cls_idx, table, W1, b1, W2, b2):
  emb = _sc_gather(cls_idx.astype(jnp.int32), table)
  return _tc_mlp(emb, W1, b1, W2, b2)
